# Initial kernel scaffold; baseline (speedup 1.0000x reference)
import jax
import jax.numpy as jnp
from jax.experimental import pallas as pl


def kernel(build_order_trace, emb, W1, b1, W2, b2):
    raise NotImplementedError("write your pallas kernel here")


# speedup vs baseline: 3.3752x; 3.3752x over previous
import functools

import jax
import jax.numpy as jnp
from jax import lax
from jax.experimental import pallas as pl
from jax.experimental.pallas import tpu as pltpu
from jax.experimental.pallas import tpu_sc as plsc

VOCAB = 1000000
HID = 64
B = 16384
L = 200

NTILES = 32
RPT = B // NTILES
WIN = 100
WPR = L // WIN
G = 128
NWIN_C = G * WPR
NCH = RPT // G
RING = 4

_mesh = plsc.VectorSubcoreMesh(core_axis_name="c", subcore_axis_name="s")


@functools.partial(
    pl.kernel,
    mesh=_mesh,
    out_type=jax.ShapeDtypeStruct((B, HID), jnp.float32),
    scratch_types=[
        pltpu.VMEM((NWIN_C, WIN), jnp.int32),
        pltpu.VMEM((RING, WIN, HID), jnp.float32),
        pltpu.VMEM((G, HID), jnp.float32),
        pltpu.SemaphoreType.DMA,
        pltpu.SemaphoreType.DMA,
        pltpu.SemaphoreType.DMA,
        pltpu.SemaphoreType.DMA,
    ],
    compiler_params=pltpu.CompilerParams(use_tc_tiling_on_sc=False),
)
def _gather_pool(ids_hbm, emb_hbm, out_hbm, idx_v, rows_v, out_v, s0, s1, s2, s3):
    sems = (s0, s1, s2, s3)
    wid = lax.axis_index("s") * 2 + lax.axis_index("c")
    row0 = wid * RPT

    def _accum_window(b, acc):
        def body(i, acc):
            a0, a1, a2, a3 = acc
            a0 = a0 + rows_v[b, i, pl.ds(0, 16)]
            a1 = a1 + rows_v[b, i, pl.ds(16, 16)]
            a2 = a2 + rows_v[b, i, pl.ds(32, 16)]
            a3 = a3 + rows_v[b, i, pl.ds(48, 16)]
            return (a0, a1, a2, a3)
        return lax.fori_loop(0, WIN, body, acc)

    @pl.loop(0, NCH)
    def _chunk(c):
        base = row0 + c * G
        pltpu.sync_copy(ids_hbm.at[pl.ds(base * WPR, NWIN_C)], idx_v)
        for b in range(RING):
            pltpu.async_copy(emb_hbm.at[idx_v.at[b]], rows_v.at[b], sems[b])

        @pl.loop(0, NWIN_C, step=RING)
        def _group(w):
            for pair in range(RING // WPR):
                r = w // WPR + pair
                zero = jnp.zeros((16,), jnp.float32)
                acc = (zero, zero, zero, zero)
                for h in range(WPR):
                    b = pair * WPR + h
                    pltpu.make_async_copy(
                        emb_hbm.at[idx_v.at[w + b]], rows_v.at[b], sems[b]
                    ).wait()
                    acc = _accum_window(b, acc)

                    @pl.when(w + RING + b < NWIN_C)
                    def _refire():
                        pltpu.async_copy(
                            emb_hbm.at[idx_v.at[w + RING + b]], rows_v.at[b], sems[b]
                        )
                out_v[r, pl.ds(0, 16)] = acc[0]
                out_v[r, pl.ds(16, 16)] = acc[1]
                out_v[r, pl.ds(32, 16)] = acc[2]
                out_v[r, pl.ds(48, 16)] = acc[3]

        pltpu.sync_copy(out_v, out_hbm.at[pl.ds(base, G)])


def _erf(x):
    a1, a2, a3, a4, a5 = (
        0.254829592, -0.284496736, 1.421413741, -1.453152027, 1.061405429)
    p = 0.3275911
    s = jnp.sign(x)
    ax = jnp.abs(x)
    t = 1.0 / (1.0 + p * ax)
    poly = t * (a1 + t * (a2 + t * (a3 + t * (a4 + t * a5))))
    return s * (1.0 - poly * jnp.exp(-ax * ax))


def _gelu(x):
    return 0.5 * x * (1.0 + _erf(x * jnp.float32(0.7071067811865476)))


def _mlp_body(x_ref, w1t_ref, b1_ref, w2t_ref, b2_ref, o_ref):
    x = x_ref[...] / jnp.float32(float(L))
    h = _gelu(jnp.dot(x, w1t_ref[...], preferred_element_type=jnp.float32)
              + b1_ref[...])
    o_ref[...] = _gelu(jnp.dot(h, w2t_ref[...], preferred_element_type=jnp.float32)
                       + b2_ref[...])


_BM = 4096


def _mlp(pooled_sum, w1t, b1, w2t, b2):
    grid = (B // _BM,)
    return pl.pallas_call(
        _mlp_body,
        grid=grid,
        in_specs=[
            pl.BlockSpec((_BM, HID), lambda i: (i, 0)),
            pl.BlockSpec((HID, HID), lambda i: (0, 0)),
            pl.BlockSpec((1, HID), lambda i: (0, 0)),
            pl.BlockSpec((HID, HID), lambda i: (0, 0)),
            pl.BlockSpec((1, HID), lambda i: (0, 0)),
        ],
        out_specs=pl.BlockSpec((_BM, HID), lambda i: (i, 0)),
        out_shape=jax.ShapeDtypeStruct((B, HID), jnp.float32),
    )(pooled_sum, w1t, b1, w2t, b2)


def kernel(build_order_trace, emb, W1, b1, W2, b2):
    ids_p1 = (build_order_trace.astype(jnp.int32) + 1).reshape(B * WPR, WIN)
    pooled_sum = _gather_pool(ids_p1, emb)
    return _mlp(pooled_sum, W1.T, b1.reshape(1, HID), W2.T, b2.reshape(1, HID))
